# Initial kernel scaffold; baseline (speedup 1.0000x reference)
import jax
import jax.numpy as jnp
from jax.experimental import pallas as pl


def kernel(Q, K, V):
    raise NotImplementedError("write your pallas kernel here")


# speedup vs baseline: 20.2346x; 20.2346x over previous
import functools

import jax
import jax.numpy as jnp
from jax.experimental import pallas as pl
from jax.experimental.pallas import tpu as pltpu

_DK = 64
_K = 32
_SEQ = 2048
_LANES = 128
_DEPTH = _SEQ // _LANES
_ROWS = 256
_SCALE = 0.125


def _batcher_pairs(n):
    pairs = []
    p = 1
    while p < n:
        k = p
        while k >= 1:
            for j in range(k % p, n - k, 2 * k):
                for i in range(0, min(k, n - j - k)):
                    if (i + j) // (p * 2) == (i + j + k) // (p * 2):
                        pairs.append((i + j, i + j + k))
            k //= 2
        p *= 2
    return pairs


_PAIRS = _batcher_pairs(_DEPTH)


def _attn_kernel(q_ref, k_ref, v_ref, attn_ref, ctx_ref):
    q = q_ref[0]
    k = k_ref[0]
    v = v_ref[0]

    scores = jax.lax.dot_general(
        q, k, (((1,), (1,)), ((), ())), preferred_element_type=jnp.float32
    ) * _SCALE

    s = [scores[:, d * _LANES:(d + 1) * _LANES] for d in range(_DEPTH)]

    for a, b in _PAIRS:
        hi = jnp.maximum(s[a], s[b])
        lo = jnp.minimum(s[a], s[b])
        s[a], s[b] = hi, lo

    neg = jnp.float32(-jnp.inf)
    thresh = None
    for it in range(_K):
        m = jnp.max(s[0], axis=-1, keepdims=True)
        thresh = m
        if it == _K - 1:
            break
        adv = s[0] == m
        for d in range(_DEPTH - 1):
            s[d] = jnp.where(adv, s[d + 1], s[d])
        s[_DEPTH - 1] = jnp.where(adv, neg, s[_DEPTH - 1])

    sel = scores >= thresh
    sparse = jnp.where(sel, jnp.exp(scores), jnp.float32(1.0))
    denom = jnp.sum(sparse, axis=-1, keepdims=True) + jnp.float32(1e-8)
    attn = sparse / denom
    attn_ref[0] = attn
    ctx_ref[0] = jax.lax.dot_general(
        attn, v, (((1,), (0,)), ((), ())), preferred_element_type=jnp.float32
    )


@functools.partial(jax.jit, static_argnames=("interpret",))
def _run(Q, K, V, interpret=False):
    B, H, S, D = Q.shape
    q3 = Q.reshape(H, S, D)
    k3 = K.reshape(H, S, D)
    v3 = V.reshape(H, S, D)
    n_rb = S // _ROWS
    grid = (H, n_rb)
    attn, ctx = pl.pallas_call(
        _attn_kernel,
        grid=grid,
        in_specs=[
            pl.BlockSpec((1, _ROWS, D), lambda h, r: (h, r, 0)),
            pl.BlockSpec((1, S, D), lambda h, r: (h, 0, 0)),
            pl.BlockSpec((1, S, D), lambda h, r: (h, 0, 0)),
        ],
        out_specs=[
            pl.BlockSpec((1, _ROWS, S), lambda h, r: (h, r, 0)),
            pl.BlockSpec((1, _ROWS, D), lambda h, r: (h, r, 0)),
        ],
        out_shape=[
            jax.ShapeDtypeStruct((H, S, S), jnp.float32),
            jax.ShapeDtypeStruct((H, S, D), jnp.float32),
        ],
        compiler_params=pltpu.CompilerParams(
            dimension_semantics=("parallel", "parallel"),
        ),
        interpret=interpret,
    )(q3, k3, v3)
    return ctx.reshape(B, H, S, D), attn.reshape(B, H, S, S)


def kernel(Q, K, V):
    context, attn = _run(Q, K, V)
    return (context, attn)


# speedup vs baseline: 23.2673x; 1.1499x over previous
import functools

import jax
import jax.numpy as jnp
from jax.experimental import pallas as pl
from jax.experimental.pallas import tpu as pltpu

_DK = 64
_K = 32
_SEQ = 2048
_LANES = 128
_DEPTH = _SEQ // _LANES
_ROWS = 256
_SCALE = 0.125


def _batcher_pairs(n):
    pairs = []
    p = 1
    while p < n:
        k = p
        while k >= 1:
            for j in range(k % p, n - k, 2 * k):
                for i in range(0, min(k, n - j - k)):
                    if (i + j) // (p * 2) == (i + j + k) // (p * 2):
                        pairs.append((i + j, i + j + k))
            k //= 2
        p *= 2
    return pairs


_PAIRS = _batcher_pairs(_DEPTH)


def _attn_kernel(q_ref, k_ref, v_ref, attn_ref, ctx_ref):
    q = q_ref[0]
    k = k_ref[0]
    v = v_ref[0]

    scores = jax.lax.dot_general(
        q, k, (((1,), (1,)), ((), ())), preferred_element_type=jnp.float32
    ) * _SCALE

    s = [scores[:, d * _LANES:(d + 1) * _LANES] for d in range(_DEPTH)]

    for a, b in _PAIRS:
        hi = jnp.maximum(s[a], s[b])
        lo = jnp.minimum(s[a], s[b])
        s[a], s[b] = hi, lo

    neg = jnp.float32(-jnp.inf)
    head = s[0]
    dep = jnp.zeros(head.shape, jnp.int32)
    thresh = None
    for it in range(_K):
        m = jnp.max(head, axis=-1, keepdims=True)
        thresh = m
        if it == _K - 1:
            break
        adv = head == m
        dep = dep + adv.astype(jnp.int32)
        w = min(it + 2, _DEPTH)
        level = list(s[:w])
        bit = 0
        while len(level) > 1:
            b = (dep & (1 << bit)) != 0
            nxt = []
            for i in range(0, len(level) - 1, 2):
                nxt.append(jnp.where(b, level[i + 1], level[i]))
            if len(level) % 2 == 1:
                nxt.append(level[-1])
            level = nxt
            bit += 1
        fetch = level[0]
        if it + 1 >= _DEPTH:
            fetch = jnp.where(dep >= _DEPTH, neg, fetch)
        head = jnp.where(adv, fetch, head)

    sel = scores >= thresh
    sparse = jnp.where(sel, jnp.exp(scores), jnp.float32(1.0))
    denom = jnp.sum(sparse, axis=-1, keepdims=True) + jnp.float32(1e-8)
    attn = sparse / denom
    attn_ref[0] = attn
    ctx_ref[0] = jax.lax.dot_general(
        attn, v, (((1,), (0,)), ((), ())), preferred_element_type=jnp.float32
    )


@functools.partial(jax.jit, static_argnames=("interpret",))
def _run(Q, K, V, interpret=False):
    B, H, S, D = Q.shape
    q3 = Q.reshape(H, S, D)
    k3 = K.reshape(H, S, D)
    v3 = V.reshape(H, S, D)
    n_rb = S // _ROWS
    grid = (H, n_rb)
    attn, ctx = pl.pallas_call(
        _attn_kernel,
        grid=grid,
        in_specs=[
            pl.BlockSpec((1, _ROWS, D), lambda h, r: (h, r, 0)),
            pl.BlockSpec((1, S, D), lambda h, r: (h, 0, 0)),
            pl.BlockSpec((1, S, D), lambda h, r: (h, 0, 0)),
        ],
        out_specs=[
            pl.BlockSpec((1, _ROWS, S), lambda h, r: (h, r, 0)),
            pl.BlockSpec((1, _ROWS, D), lambda h, r: (h, r, 0)),
        ],
        out_shape=[
            jax.ShapeDtypeStruct((H, S, S), jnp.float32),
            jax.ShapeDtypeStruct((H, S, D), jnp.float32),
        ],
        compiler_params=pltpu.CompilerParams(
            dimension_semantics=("parallel", "parallel"),
        ),
        interpret=interpret,
    )(q3, k3, v3)
    return ctx.reshape(B, H, S, D), attn.reshape(B, H, S, S)


def kernel(Q, K, V):
    context, attn = _run(Q, K, V)
    return (context, attn)


# speedup vs baseline: 25.8926x; 1.1128x over previous
import functools

import jax
import jax.numpy as jnp
from jax.experimental import pallas as pl
from jax.experimental.pallas import tpu as pltpu

_DK = 64
_K = 32
_SEQ = 2048
_LANES = 128
_DEPTH = _SEQ // _LANES
_CAP = 7
_ROWS = 256
_SCALE = 0.125


def _batcher_pairs(n):
    pairs = []
    p = 1
    while p < n:
        k = p
        while k >= 1:
            for j in range(k % p, n - k, 2 * k):
                for i in range(0, min(k, n - j - k)):
                    if (i + j) // (p * 2) == (i + j + k) // (p * 2):
                        pairs.append((i + j, i + j + k))
            k //= 2
        p *= 2
    return pairs


def _pruned_pairs(pairs, top):
    needed = set(range(top))
    kept = []
    for a, b in reversed(pairs):
        if a in needed or b in needed:
            kept.append((a, b))
            needed.add(a)
            needed.add(b)
    kept.reverse()
    return kept


_PAIRS = _batcher_pairs(_DEPTH)
_PAIRS_TOP = _pruned_pairs(_PAIRS, _CAP)


def _col_sort(scores, pairs):
    s = [scores[:, d * _LANES:(d + 1) * _LANES] for d in range(_DEPTH)]
    for a, b in pairs:
        hi = jnp.maximum(s[a], s[b])
        lo = jnp.minimum(s[a], s[b])
        s[a], s[b] = hi, lo
    return s


def _extract(s, cap, want_viol):
    neg = jnp.float32(-jnp.inf)
    head = s[0]
    dep = jnp.zeros(head.shape, jnp.int32)
    thresh = None
    for it in range(_K):
        m = jnp.max(head, axis=-1, keepdims=True)
        thresh = m
        if it == _K - 1:
            break
        adv = head == m
        dep = dep + adv.astype(jnp.int32)
        w = min(it + 2, cap)
        level = list(s[:w])
        bit = 0
        while len(level) > 1:
            b = (dep & (1 << bit)) != 0
            nxt = []
            for i in range(0, len(level) - 1, 2):
                nxt.append(jnp.where(b, level[i + 1], level[i]))
            if len(level) % 2 == 1:
                nxt.append(level[-1])
            level = nxt
            bit += 1
        fetch = level[0]
        if cap == _DEPTH and it + 1 >= _DEPTH:
            fetch = jnp.where(dep >= _DEPTH, neg, fetch)
        head = jnp.where(adv, fetch, head)
    viol = None
    if want_viol:
        viol = jnp.max(dep) > cap - 1
    return thresh, viol


def _attn_kernel(q_ref, k_ref, v_ref, attn_ref, ctx_ref, thr_ref):
    q = q_ref[0] * jnp.float32(_SCALE)
    k = k_ref[0]
    v = v_ref[0]

    scores = jax.lax.dot_general(
        q, k, (((1,), (1,)), ((), ())), preferred_element_type=jnp.float32
    )

    s = _col_sort(scores, _PAIRS_TOP)
    thresh, viol = _extract(s, _CAP, True)
    thr_ref[...] = thresh

    @pl.when(viol)
    def _fallback():
        s_full = _col_sort(scores, _PAIRS)
        t_full, _ = _extract(s_full, _DEPTH, False)
        thr_ref[...] = t_full

    t = thr_ref[...]
    sel = scores >= t
    sparse = jnp.where(sel, jnp.exp(scores), jnp.float32(1.0))
    denom = jnp.sum(sparse, axis=-1, keepdims=True) + jnp.float32(1e-8)
    attn = sparse / denom
    attn_ref[0] = attn
    ctx_ref[0] = jax.lax.dot_general(
        attn, v, (((1,), (0,)), ((), ())), preferred_element_type=jnp.float32
    )


@functools.partial(jax.jit, static_argnames=("interpret",))
def _run(Q, K, V, interpret=False):
    B, H, S, D = Q.shape
    q3 = Q.reshape(H, S, D)
    k3 = K.reshape(H, S, D)
    v3 = V.reshape(H, S, D)
    n_rb = S // _ROWS
    grid = (H, n_rb)
    attn, ctx = pl.pallas_call(
        _attn_kernel,
        grid=grid,
        in_specs=[
            pl.BlockSpec((1, _ROWS, D), lambda h, r: (h, r, 0)),
            pl.BlockSpec((1, S, D), lambda h, r: (h, 0, 0)),
            pl.BlockSpec((1, S, D), lambda h, r: (h, 0, 0)),
        ],
        out_specs=[
            pl.BlockSpec((1, _ROWS, S), lambda h, r: (h, r, 0)),
            pl.BlockSpec((1, _ROWS, D), lambda h, r: (h, r, 0)),
        ],
        out_shape=[
            jax.ShapeDtypeStruct((H, S, S), jnp.float32),
            jax.ShapeDtypeStruct((H, S, D), jnp.float32),
        ],
        scratch_shapes=[pltpu.VMEM((_ROWS, 1), jnp.float32)],
        compiler_params=pltpu.CompilerParams(
            dimension_semantics=("parallel", "parallel"),
        ),
        interpret=interpret,
    )(q3, k3, v3)
    return ctx.reshape(B, H, S, D), attn.reshape(B, H, S, S)


def kernel(Q, K, V):
    context, attn = _run(Q, K, V)
    return (context, attn)


# speedup vs baseline: 30.2111x; 1.1668x over previous
import functools

import jax
import jax.numpy as jnp
from jax.experimental import pallas as pl
from jax.experimental.pallas import tpu as pltpu

_DK = 64
_K = 32
_SEQ = 2048
_LANES = 128
_DEPTH = _SEQ // _LANES
_CAP = 7
_ROWS = 512
_SCALE = 0.125


def _batcher_pairs(n):
    pairs = []
    p = 1
    while p < n:
        k = p
        while k >= 1:
            for j in range(k % p, n - k, 2 * k):
                for i in range(0, min(k, n - j - k)):
                    if (i + j) // (p * 2) == (i + j + k) // (p * 2):
                        pairs.append((i + j, i + j + k))
            k //= 2
        p *= 2
    return pairs


def _pruned_pairs(pairs, top):
    needed = set(range(top))
    kept = []
    for a, b in reversed(pairs):
        if a in needed or b in needed:
            kept.append((a, b))
            needed.add(a)
            needed.add(b)
    kept.reverse()
    return kept


_PAIRS = _batcher_pairs(_DEPTH)
_PAIRS_TOP = _pruned_pairs(_PAIRS, _CAP)


def _col_sort(scores, pairs):
    s = [scores[:, d * _LANES:(d + 1) * _LANES] for d in range(_DEPTH)]
    for a, b in pairs:
        hi = jnp.maximum(s[a], s[b])
        lo = jnp.minimum(s[a], s[b])
        s[a], s[b] = hi, lo
    return s


def _extract(s, cap, want_viol):
    neg = jnp.float32(-jnp.inf)
    head = s[0]
    dep = jnp.zeros(head.shape, jnp.int32)
    thresh = None
    for it in range(_K):
        m = jnp.max(head, axis=-1, keepdims=True)
        thresh = m
        if it == _K - 1:
            break
        adv = head == m
        dep = dep + adv.astype(jnp.int32)
        w = min(it + 2, cap)
        level = list(s[:w])
        bit = 0
        while len(level) > 1:
            b = (dep & (1 << bit)) != 0
            nxt = []
            for i in range(0, len(level) - 1, 2):
                nxt.append(jnp.where(b, level[i + 1], level[i]))
            if len(level) % 2 == 1:
                nxt.append(level[-1])
            level = nxt
            bit += 1
        fetch = level[0]
        if cap == _DEPTH and it + 1 >= _DEPTH:
            fetch = jnp.where(dep >= _DEPTH, neg, fetch)
        head = jnp.where(adv, fetch, head)
    viol = None
    if want_viol:
        viol = jnp.max(dep) > cap - 1
    return thresh, viol


def _attn_kernel(q_ref, k_ref, v_ref, attn_ref, ctx_ref, thr_ref):
    q = q_ref[0] * jnp.float32(_SCALE)
    k = k_ref[0]
    v = v_ref[0]

    scores = jax.lax.dot_general(
        q, k, (((1,), (1,)), ((), ())), preferred_element_type=jnp.float32
    )

    s = _col_sort(scores, _PAIRS_TOP)
    thresh, viol = _extract(s, _CAP, True)
    thr_ref[...] = thresh

    @pl.when(viol)
    def _fallback():
        s_full = _col_sort(scores, _PAIRS)
        t_full, _ = _extract(s_full, _DEPTH, False)
        thr_ref[...] = t_full

    t = thr_ref[...]
    sel = scores >= t
    sparse = jnp.where(sel, jnp.exp(scores), jnp.float32(1.0))
    denom = jnp.sum(sparse, axis=-1, keepdims=True) + jnp.float32(1e-8)
    attn = sparse / denom
    attn_ref[0] = attn
    ctx_ref[0] = jax.lax.dot_general(
        attn, v, (((1,), (0,)), ((), ())), preferred_element_type=jnp.float32
    )


@functools.partial(jax.jit, static_argnames=("interpret",))
def _run(Q, K, V, interpret=False):
    B, H, S, D = Q.shape
    q3 = Q.reshape(H, S, D)
    k3 = K.reshape(H, S, D)
    v3 = V.reshape(H, S, D)
    n_rb = S // _ROWS
    grid = (H, n_rb)
    attn, ctx = pl.pallas_call(
        _attn_kernel,
        grid=grid,
        in_specs=[
            pl.BlockSpec((1, _ROWS, D), lambda h, r: (h, r, 0)),
            pl.BlockSpec((1, S, D), lambda h, r: (h, 0, 0)),
            pl.BlockSpec((1, S, D), lambda h, r: (h, 0, 0)),
        ],
        out_specs=[
            pl.BlockSpec((1, _ROWS, S), lambda h, r: (h, r, 0)),
            pl.BlockSpec((1, _ROWS, D), lambda h, r: (h, r, 0)),
        ],
        out_shape=[
            jax.ShapeDtypeStruct((H, S, S), jnp.float32),
            jax.ShapeDtypeStruct((H, S, D), jnp.float32),
        ],
        scratch_shapes=[pltpu.VMEM((_ROWS, 1), jnp.float32)],
        compiler_params=pltpu.CompilerParams(
            dimension_semantics=("parallel", "parallel"),
        ),
        interpret=interpret,
    )(q3, k3, v3)
    return ctx.reshape(B, H, S, D), attn.reshape(B, H, S, S)


def kernel(Q, K, V):
    context, attn = _run(Q, K, V)
    return (context, attn)


# speedup vs baseline: 30.4263x; 1.0071x over previous
import functools

import jax
import jax.numpy as jnp
from jax.experimental import pallas as pl
from jax.experimental.pallas import tpu as pltpu

_DK = 64
_K = 32
_SEQ = 2048
_LANES = 128
_DEPTH = _SEQ // _LANES
_CAP = 7
_ROWS = 1024
_SCALE = 0.125


def _batcher_pairs(n):
    pairs = []
    p = 1
    while p < n:
        k = p
        while k >= 1:
            for j in range(k % p, n - k, 2 * k):
                for i in range(0, min(k, n - j - k)):
                    if (i + j) // (p * 2) == (i + j + k) // (p * 2):
                        pairs.append((i + j, i + j + k))
            k //= 2
        p *= 2
    return pairs


def _pruned_pairs(pairs, top):
    needed = set(range(top))
    kept = []
    for a, b in reversed(pairs):
        if a in needed or b in needed:
            kept.append((a, b))
            needed.add(a)
            needed.add(b)
    kept.reverse()
    return kept


_PAIRS = _batcher_pairs(_DEPTH)
_PAIRS_TOP = _pruned_pairs(_PAIRS, _CAP)


def _col_sort(scores, pairs):
    s = [scores[:, d * _LANES:(d + 1) * _LANES] for d in range(_DEPTH)]
    for a, b in pairs:
        hi = jnp.maximum(s[a], s[b])
        lo = jnp.minimum(s[a], s[b])
        s[a], s[b] = hi, lo
    return s


def _extract(s, cap, want_viol):
    neg = jnp.float32(-jnp.inf)
    head = s[0]
    dep = jnp.zeros(head.shape, jnp.int32)
    thresh = None
    for it in range(_K):
        m = jnp.max(head, axis=-1, keepdims=True)
        thresh = m
        if it == _K - 1:
            break
        adv = head == m
        dep = dep + adv.astype(jnp.int32)
        w = min(it + 2, cap)
        level = list(s[:w])
        bit = 0
        while len(level) > 1:
            b = (dep & (1 << bit)) != 0
            nxt = []
            for i in range(0, len(level) - 1, 2):
                nxt.append(jnp.where(b, level[i + 1], level[i]))
            if len(level) % 2 == 1:
                nxt.append(level[-1])
            level = nxt
            bit += 1
        fetch = level[0]
        if cap == _DEPTH and it + 1 >= _DEPTH:
            fetch = jnp.where(dep >= _DEPTH, neg, fetch)
        head = jnp.where(adv, fetch, head)
    viol = None
    if want_viol:
        viol = jnp.max(dep) > cap - 1
    return thresh, viol


def _attn_kernel(q_ref, k_ref, v_ref, attn_ref, ctx_ref, thr_ref):
    q = q_ref[0] * jnp.float32(_SCALE)
    k = k_ref[0]
    v = v_ref[0]

    scores = jax.lax.dot_general(
        q, k, (((1,), (1,)), ((), ())), preferred_element_type=jnp.float32
    )

    s = _col_sort(scores, _PAIRS_TOP)
    thresh, viol = _extract(s, _CAP, True)
    thr_ref[...] = thresh

    @pl.when(viol)
    def _fallback():
        s_full = _col_sort(scores, _PAIRS)
        t_full, _ = _extract(s_full, _DEPTH, False)
        thr_ref[...] = t_full

    t = thr_ref[...]
    sel = scores >= t
    sparse = jnp.where(sel, jnp.exp(scores), jnp.float32(1.0))
    denom = jnp.sum(sparse, axis=-1, keepdims=True) + jnp.float32(1e-8)
    attn = sparse / denom
    attn_ref[0] = attn
    ctx_ref[0] = jax.lax.dot_general(
        attn, v, (((1,), (0,)), ((), ())), preferred_element_type=jnp.float32
    )


@functools.partial(jax.jit, static_argnames=("interpret",))
def _run(Q, K, V, interpret=False):
    B, H, S, D = Q.shape
    q3 = Q.reshape(H, S, D)
    k3 = K.reshape(H, S, D)
    v3 = V.reshape(H, S, D)
    n_rb = S // _ROWS
    grid = (H, n_rb)
    attn, ctx = pl.pallas_call(
        _attn_kernel,
        grid=grid,
        in_specs=[
            pl.BlockSpec((1, _ROWS, D), lambda h, r: (h, r, 0)),
            pl.BlockSpec((1, S, D), lambda h, r: (h, 0, 0)),
            pl.BlockSpec((1, S, D), lambda h, r: (h, 0, 0)),
        ],
        out_specs=[
            pl.BlockSpec((1, _ROWS, S), lambda h, r: (h, r, 0)),
            pl.BlockSpec((1, _ROWS, D), lambda h, r: (h, r, 0)),
        ],
        out_shape=[
            jax.ShapeDtypeStruct((H, S, S), jnp.float32),
            jax.ShapeDtypeStruct((H, S, D), jnp.float32),
        ],
        scratch_shapes=[pltpu.VMEM((_ROWS, 1), jnp.float32)],
        compiler_params=pltpu.CompilerParams(
            dimension_semantics=("parallel", "parallel"),
        ),
        interpret=interpret,
    )(q3, k3, v3)
    return ctx.reshape(B, H, S, D), attn.reshape(B, H, S, S)


def kernel(Q, K, V):
    context, attn = _run(Q, K, V)
    return (context, attn)
